# baseline (speedup 1.0000x reference)
import functools

import jax
import jax.numpy as jnp
from jax import lax
from jax.experimental import pallas as pl
from jax.experimental.pallas import tpu as pltpu
from jax.experimental.pallas import tpu_sc as plsc

NBUF = 2


def _embed_kernel(B, H, V, D, nc, ns):
    NW = nc * ns
    KI = B // 128
    n_blocks = H * KI
    per_w = n_blocks // NW
    CB = D // 8

    mesh = plsc.VectorSubcoreMesh(core_axis_name="c", subcore_axis_name="s")

    @functools.partial(
        pl.kernel,
        mesh=mesh,
        compiler_params=pltpu.CompilerParams(
            use_tc_tiling_on_sc=False,
            skip_device_barrier=True,
            needs_layout_passes=False,
        ),
        out_type=jax.ShapeDtypeStruct((H, CB, KI, 8, 128), jnp.float32),
        scratch_types=[
            pltpu.VMEM((NBUF, 128), jnp.int32),
            pltpu.VMEM((NBUF, 128, D), jnp.float32),
            pltpu.VMEM((NBUF, D, 128), jnp.float32),
            pltpu.SemaphoreType.DMA,
            pltpu.SemaphoreType.DMA,
            pltpu.SemaphoreType.DMA,
        ],
    )
    def k(x4, table_hbm, p5, idx_v, rows_v, tile_v, sem_i, sem_g, sem_o):
        wid = lax.axis_index("s") * nc + lax.axis_index("c")
        t0 = wid * per_w

        row_idx = [
            lax.iota(jnp.int32, 16) + jnp.int32(16 * g) for g in range(8)
        ]

        def fire_idx(t, b):
            h = t // KI
            kk = lax.rem(t, KI)
            pltpu.async_copy(x4.at[h // 8, kk, lax.rem(h, 8)], idx_v.at[b], sem_i)

        def wait_idx(b):
            pltpu.make_async_copy(x4.at[0, 0, 0], idx_v.at[b], sem_i).wait()

        def fire_gather(b):
            pltpu.async_copy(table_hbm.at[idx_v.at[b]], rows_v.at[b], sem_g)

        def wait_gather(b):
            pltpu.make_async_copy(
                table_hbm.at[pl.ds(0, 128)], rows_v.at[b], sem_g
            ).wait()

        def transpose(b):
            for c in range(D):
                cvec = jnp.full((16,), c, dtype=jnp.int32)
                for g in range(8):
                    v = plsc.load_gather(rows_v.at[b], [row_idx[g], cvec])
                    tile_v[b, c, pl.ds(16 * g, 16)] = v

        def fire_store(t, b):
            h = t // KI
            kk = lax.rem(t, KI)
            for cb in range(CB):
                pltpu.async_copy(
                    tile_v.at[b, pl.ds(8 * cb, 8)], p5.at[h, cb, kk], sem_o
                )

        def wait_store(b):
            for cb in range(CB):
                pltpu.make_async_copy(
                    tile_v.at[b, pl.ds(8 * cb, 8)], p5.at[0, cb, 0], sem_o
                ).wait()

        fire_idx(t0, 0)
        fire_idx(t0 + 1, 1)
        wait_idx(0)
        fire_gather(0)

        @pl.loop(0, per_w, step=NBUF)
        def _(j0):
            for u in range(NBUF):
                j = j0 + u
                t = t0 + j
                b = u
                b1 = (u + 1) % NBUF
                @pl.when(j + 1 < per_w)
                def _():
                    wait_idx(b1)
                    fire_gather(b1)
                wait_gather(b)
                @pl.when(j >= NBUF)
                def _():
                    wait_store(b)
                transpose(b)
                fire_store(t, b)
                @pl.when(j + NBUF < per_w)
                def _():
                    fire_idx(t + NBUF, b)

        wait_store(0)
        wait_store(1)

    return k


def kernel(x, table):
    B, H = x.shape
    V, D = table.shape
    x4 = jnp.transpose(jnp.transpose(x).reshape(H // 8, 8, B // 128, 128), (0, 2, 1, 3))
    info = plsc.get_sparse_core_info()
    p5 = _embed_kernel(B, H, V, D, info.num_cores, info.num_subcores)(x4, table)
    o = jnp.transpose(p5, (2, 4, 0, 1, 3))
    return o.reshape(B, H, D)


# speedup vs baseline: 1.0009x; 1.0009x over previous
import functools

import jax
import jax.numpy as jnp
from jax import lax
from jax.experimental import pallas as pl
from jax.experimental.pallas import tpu as pltpu
from jax.experimental.pallas import tpu_sc as plsc

NBUF = 2


def _embed_kernel(B, H, V, D, nc, ns):
    NW = nc * ns
    KI = B // 128
    n_blocks = H * KI
    per_w = n_blocks // NW
    CB = D // 8

    mesh = plsc.VectorSubcoreMesh(core_axis_name="c", subcore_axis_name="s")

    @functools.partial(
        pl.kernel,
        mesh=mesh,
        compiler_params=pltpu.CompilerParams(
            use_tc_tiling_on_sc=False,
            skip_device_barrier=True,
            needs_layout_passes=False,
            disable_bounds_checks=True,
        ),
        out_type=jax.ShapeDtypeStruct((H, CB, KI, 8, 128), jnp.float32),
        scratch_types=[
            pltpu.VMEM((NBUF, 128), jnp.int32),
            pltpu.VMEM((NBUF, 128, D), jnp.float32),
            pltpu.VMEM((NBUF, D, 128), jnp.float32),
            pltpu.SemaphoreType.DMA,
            pltpu.SemaphoreType.DMA,
            pltpu.SemaphoreType.DMA,
        ],
    )
    def k(xt, table_hbm, p5, idx_v, rows_v, tile_v, sem_i, sem_g, sem_o):
        wid = lax.axis_index("s") * nc + lax.axis_index("c")
        t0 = wid * per_w

        row_idx = [
            lax.iota(jnp.int32, 16) + jnp.int32(16 * g) for g in range(8)
        ]

        def fire_idx(t, b):
            h = t // KI
            kk = lax.rem(t, KI)
            pltpu.async_copy(xt.at[h, pl.ds(128 * kk, 128)], idx_v.at[b], sem_i)

        def wait_idx(b):
            pltpu.make_async_copy(xt.at[0, pl.ds(0, 128)], idx_v.at[b], sem_i).wait()

        def fire_gather(b):
            pltpu.async_copy(table_hbm.at[idx_v.at[b]], rows_v.at[b], sem_g)

        def wait_gather(b):
            pltpu.make_async_copy(
                table_hbm.at[pl.ds(0, 128)], rows_v.at[b], sem_g
            ).wait()

        def transpose(b):
            for c in range(D):
                cvec = jnp.full((16,), c, dtype=jnp.int32)
                for g in range(8):
                    v = plsc.load_gather(rows_v.at[b], [row_idx[g], cvec])
                    tile_v[b, c, pl.ds(16 * g, 16)] = v

        def fire_store(t, b):
            h = t // KI
            kk = lax.rem(t, KI)
            for cb in range(CB):
                pltpu.async_copy(
                    tile_v.at[b, pl.ds(8 * cb, 8)], p5.at[h, cb, kk], sem_o
                )

        def wait_store(b):
            for cb in range(CB):
                pltpu.make_async_copy(
                    tile_v.at[b, pl.ds(8 * cb, 8)], p5.at[0, cb, 0], sem_o
                ).wait()

        fire_idx(t0, 0)
        fire_idx(t0 + 1, 1)
        wait_idx(0)
        fire_gather(0)

        @pl.loop(0, per_w, step=NBUF)
        def _(j0):
            for u in range(NBUF):
                j = j0 + u
                t = t0 + j
                b = u
                b1 = (u + 1) % NBUF
                @pl.when(j + 1 < per_w)
                def _():
                    wait_idx(b1)
                    fire_gather(b1)
                wait_gather(b)
                @pl.when(j >= NBUF)
                def _():
                    wait_store(b)
                transpose(b)
                fire_store(t, b)
                @pl.when(j + NBUF < per_w)
                def _():
                    fire_idx(t + NBUF, b)

        wait_store(0)
        wait_store(1)

    return k


def kernel(x, table):
    B, H = x.shape
    V, D = table.shape
    xt = jnp.transpose(x)
    info = plsc.get_sparse_core_info()
    p5 = _embed_kernel(B, H, V, D, info.num_cores, info.num_subcores)(xt, table)
    o = jnp.transpose(p5, (2, 4, 0, 1, 3))
    return o.reshape(B, H, D)


# speedup vs baseline: 1.9436x; 1.9419x over previous
import functools

import jax
import jax.numpy as jnp
from jax import lax
from jax.experimental import pallas as pl
from jax.experimental.pallas import tpu as pltpu
from jax.experimental.pallas import tpu_sc as plsc

NBUF = 2


def _detile_x(H, B):
    KI = B // 128

    def body(x_ref, y_ref):
        for s in range(8):
            for kk in range(KI):
                y_ref[KI * s + kk, :] = x_ref[s, pl.ds(128 * kk, 128)]

    return pl.pallas_call(
        body,
        grid=(H // 8,),
        in_specs=[pl.BlockSpec((8, B), lambda g: (g, 0))],
        out_specs=pl.BlockSpec((8 * KI, 128), lambda g: (g, 0)),
        out_shape=jax.ShapeDtypeStruct((H * KI, 128), jnp.int32),
    )


def _embed_kernel(B, H, V, D, nc, ns):
    NW = nc * ns
    KI = B // 128
    n_blocks = H * KI
    per_w = n_blocks // NW
    CB = D // 8

    mesh = plsc.VectorSubcoreMesh(core_axis_name="c", subcore_axis_name="s")

    @functools.partial(
        pl.kernel,
        mesh=mesh,
        compiler_params=pltpu.CompilerParams(
            use_tc_tiling_on_sc=False,
            skip_device_barrier=True,
            needs_layout_passes=False,
            disable_bounds_checks=True,
        ),
        out_type=jax.ShapeDtypeStruct((H, CB, KI, 8, 128), jnp.float32),
        scratch_types=[
            pltpu.VMEM((NBUF, 128), jnp.int32),
            pltpu.VMEM((NBUF, 128, D), jnp.float32),
            pltpu.VMEM((NBUF, D, 128), jnp.float32),
            pltpu.SemaphoreType.DMA,
            pltpu.SemaphoreType.DMA,
            pltpu.SemaphoreType.DMA,
        ],
    )
    def k(y, table_hbm, p5, idx_v, rows_v, tile_v, sem_i, sem_g, sem_o):
        wid = lax.axis_index("s") * nc + lax.axis_index("c")
        t0 = wid * per_w

        iota = lax.iota(jnp.int32, 16)
        row16 = [iota + jnp.int32(16 * g) for g in range(8)]
        pvec = [lax.rem(iota + jnp.int32(d), jnp.int32(16)) for d in range(16)]

        def fire_idx(t, b):
            pltpu.async_copy(y.at[t], idx_v.at[b], sem_i)

        def wait_idx(b):
            pltpu.make_async_copy(y.at[0], idx_v.at[b], sem_i).wait()

        def fire_gather(b):
            pltpu.async_copy(table_hbm.at[idx_v.at[b]], rows_v.at[b], sem_g)

        def wait_gather(b):
            pltpu.make_async_copy(
                table_hbm.at[pl.ds(0, 128)], rows_v.at[b], sem_g
            ).wait()

        def transpose(b):
            rb = rows_v.at[b]
            tb = tile_v.at[b]

            @pl.loop(0, D // 16)
            def _(cq):
                c0 = cq * 16
                for d in range(16):
                    cols = pvec[d] + c0
                    for g in range(8):
                        v = plsc.load_gather(rb, [row16[g], cols])
                        plsc.store_scatter(tb, [cols, row16[g]], v)

        def fire_store(t, b):
            h = t // KI
            kk = lax.rem(t, KI)
            for cb in range(CB):
                pltpu.async_copy(
                    tile_v.at[b, pl.ds(8 * cb, 8)], p5.at[h, cb, kk], sem_o
                )

        def wait_store(b):
            for cb in range(CB):
                pltpu.make_async_copy(
                    tile_v.at[b, pl.ds(8 * cb, 8)], p5.at[0, cb, 0], sem_o
                ).wait()

        fire_idx(t0, 0)
        fire_idx(t0 + 1, 1)
        wait_idx(0)
        fire_gather(0)

        @pl.loop(0, per_w, step=NBUF)
        def _(j0):
            for u in range(NBUF):
                j = j0 + u
                t = t0 + j
                b = u
                b1 = (u + 1) % NBUF
                @pl.when(j + 1 < per_w)
                def _():
                    wait_idx(b1)
                    fire_gather(b1)
                wait_gather(b)
                @pl.when(j >= NBUF)
                def _():
                    wait_store(b)
                transpose(b)
                fire_store(t, b)
                @pl.when(j + NBUF < per_w)
                def _():
                    fire_idx(t + NBUF, b)

        wait_store(0)
        wait_store(1)

    return k


def kernel(x, table):
    B, H = x.shape
    V, D = table.shape
    y = _detile_x(H, B)(jnp.transpose(x))
    info = plsc.get_sparse_core_info()
    p5 = _embed_kernel(B, H, V, D, info.num_cores, info.num_subcores)(y, table)
    o = jnp.transpose(p5, (2, 4, 0, 1, 3))
    return o.reshape(B, H, D)
